# baseline (speedup 1.0000x reference)
import functools

import jax
import jax.numpy as jnp
from jax import lax
from jax.experimental import pallas as pl
from jax.experimental.pallas import tpu as pltpu
from jax.experimental.pallas import tpu_sc as plsc

_BATCH = 16384
_N_FIELDS = 26
_EMBED = 16
_VOCAB = 1000000
_B = _BATCH * _N_FIELDS
_NW = 32
_B_PER_W = _B // _NW
_NCHUNK = 4
_CH = _B_PER_W // _NCHUNK

_UV = 2048
_NUNIT = _VOCAB // _UV
_TAIL0 = _NUNIT * _UV
_MID = 512
_TAIL1 = _TAIL0 + _MID

_mesh = plsc.VectorSubcoreMesh(core_axis_name="c", subcore_axis_name="s")


def _shuffle(tab_ref, og_ref, nvoc):
    i16 = lax.iota(jnp.int32, 16)

    @plsc.parallel_loop(0, nvoc // 8, unroll=8)
    def _(rr):
        for kk in range(8):
            vi = jnp.full((16,), rr * 8 + kk, jnp.int32)
            og_ref[rr, pl.ds(kk * 16, 16)] = plsc.load_gather(
                tab_ref, [i16, vi])


@functools.partial(
    pl.kernel,
    out_type=jax.ShapeDtypeStruct((_VOCAB // 8, 128), jnp.float32),
    mesh=_mesh,
    scratch_types=[
        pltpu.VMEM((16, _UV), jnp.float32),
        pltpu.VMEM((_UV // 8, 128), jnp.float32),
    ],
    compiler_params=pltpu.CompilerParams(needs_layout_passes=False),
)
def _repack_kernel(tab_hbm, tail_hbm, out_hbm, tab_v, og_v):
    wid = lax.axis_index("s") * 2 + lax.axis_index("c")

    def unit(i, _):
        u = wid + i * _NW

        @pl.when(u < _NUNIT)
        def _():
            v0 = u * _UV
            pltpu.sync_copy(tab_hbm.at[:, pl.ds(v0, _UV)], tab_v)
            _shuffle(tab_v, og_v, _UV)
            pltpu.sync_copy(og_v, out_hbm.at[pl.ds(u * (_UV // 8), _UV // 8)])

        return 0

    lax.fori_loop(0, (_NUNIT + _NW - 1) // _NW, unit, 0)

    @pl.when(wid == 1)
    def _():
        pltpu.sync_copy(tab_hbm.at[:, pl.ds(_TAIL0, _MID)],
                        tab_v.at[:, pl.ds(0, _MID)])
        _shuffle(tab_v, og_v, _MID)
        pltpu.sync_copy(og_v.at[pl.ds(0, _MID // 8)],
                        out_hbm.at[pl.ds(_TAIL0 // 8, _MID // 8)])

    @pl.when(wid == 0)
    def _():
        pltpu.sync_copy(tail_hbm, tab_v.at[:, pl.ds(0, 128)])
        _shuffle(tab_v, og_v, 128)
        pltpu.sync_copy(og_v.at[pl.ds(0, 8)],
                        out_hbm.at[pl.ds(_TAIL1 // 8, 8)])


@functools.partial(
    pl.kernel,
    out_type=jax.ShapeDtypeStruct((_B, _EMBED), jnp.float32),
    mesh=_mesh,
    scratch_types=[
        pltpu.VMEM((_CH,), jnp.int32),
        pltpu.VMEM((_CH, _EMBED), jnp.float32),
        pltpu.SemaphoreType.DMA,
    ],
    compiler_params=pltpu.CompilerParams(use_tc_tiling_on_sc=False),
)
def _gather_kernel(idx_hbm, table_hbm, out_hbm, idx_v, rows_v, sem):
    wid = lax.axis_index("s") * 2 + lax.axis_index("c")
    base = wid * _B_PER_W
    for g in range(_NCHUNK):
        off = base + g * _CH
        pltpu.sync_copy(idx_hbm.at[pl.ds(off, _CH)], idx_v)
        pltpu.async_copy(table_hbm.at[idx_v], rows_v, sem).wait()
        pltpu.sync_copy(rows_v, out_hbm.at[pl.ds(off, _CH)])


@functools.partial(
    pl.kernel,
    out_type=jax.ShapeDtypeStruct((_N_FIELDS, _EMBED, _BATCH), jnp.float32),
    mesh=_mesh,
    scratch_types=[
        pltpu.VMEM((128, 128), jnp.float32),
        pltpu.VMEM((_EMBED, 1024), jnp.float32),
    ],
    compiler_params=pltpu.CompilerParams(needs_layout_passes=False),
)
def _retile_kernel(rows_hbm, out_hbm, tin_v, och_v):
    i16 = lax.iota(jnp.int32, 16)
    hi = i16 >> 3
    si = (i16 & 7) * _EMBED
    wid = lax.axis_index("s") * 2 + lax.axis_index("c")

    def unit(c, _):
        f = c // 16
        q = c % 16
        pltpu.sync_copy(rows_hbm.at[f, pl.ds(q * 128, 128), :], tin_v)
        for j in range(_EMBED):
            colj = si + j

            @plsc.parallel_loop(0, 64, unroll=8)
            def _(k):
                rowk = hi + 2 * k
                och_v[j, pl.ds(k * 16, 16)] = plsc.load_gather(
                    tin_v, [rowk, colj])
        pltpu.sync_copy(och_v, out_hbm.at[f, :, pl.ds(q * 1024, 1024)])
        return 0

    lax.fori_loop(wid * 13, (wid + 1) * 13, unit, 0)


def kernel(index, cluster_index):
    table_t = jnp.swapaxes(cluster_index, 0, 1)
    tail = jnp.zeros((16, 128), jnp.float32)
    tail = lax.dynamic_update_slice(
        tail, lax.slice(table_t, (0, _TAIL1), (16, _VOCAB)), (0, 0))
    table_g = _repack_kernel(table_t, tail)
    table_v = jnp.reshape(table_g, (_VOCAB, _EMBED))
    flat_idx = jnp.swapaxes(index, 0, 1).reshape(-1)
    rows = _gather_kernel(flat_idx, table_v)
    rows3 = jnp.reshape(rows, (_N_FIELDS, _BATCH // 8, 128))
    out3 = _retile_kernel(rows3)
    return jnp.transpose(out3, (2, 0, 1))


# speedup vs baseline: 1.0321x; 1.0321x over previous
import functools

import jax
import jax.numpy as jnp
from jax import lax
from jax.experimental import pallas as pl
from jax.experimental.pallas import tpu as pltpu
from jax.experimental.pallas import tpu_sc as plsc

_BATCH = 16384
_N_FIELDS = 26
_EMBED = 16
_VOCAB = 1000000
_B = _BATCH * _N_FIELDS
_NW = 32
_B_PER_W = _B // _NW
_NCHUNK = 4
_CH = _B_PER_W // _NCHUNK

_UV = 2048
_NUNIT = _VOCAB // _UV
_TAIL0 = _NUNIT * _UV
_MID = 512
_TAIL1 = _TAIL0 + _MID

_mesh = plsc.VectorSubcoreMesh(core_axis_name="c", subcore_axis_name="s")


def _shuffle(tab_ref, og_ref, nvoc):
    i16 = lax.iota(jnp.int32, 16)

    @plsc.parallel_loop(0, nvoc // 8, unroll=4)
    def _(rr):
        for kk in range(8):
            vi = jnp.full((16,), rr * 8 + kk, jnp.int32)
            og_ref[rr, pl.ds(kk * 16, 16)] = plsc.load_gather(
                tab_ref, [i16, vi])


@functools.partial(
    pl.kernel,
    out_type=jax.ShapeDtypeStruct((_VOCAB // 8, 128), jnp.float32),
    mesh=_mesh,
    scratch_types=[
        pltpu.VMEM((16, _UV), jnp.float32),
        pltpu.VMEM((_UV // 8, 128), jnp.float32),
        pltpu.SemaphoreType.DMA,
    ],
    compiler_params=pltpu.CompilerParams(needs_layout_passes=False),
)
def _repack_kernel(tab_hbm, tail_hbm, out_hbm, tab_v, og_v, osem):
    wid = lax.axis_index("s") * 2 + lax.axis_index("c")

    def unit(i, _):
        u = wid + i * _NW

        @pl.when(u < _NUNIT)
        def _():
            v0 = u * _UV
            pltpu.sync_copy(tab_hbm.at[:, pl.ds(v0, _UV)], tab_v)

            @pl.when(i > 0)
            def _():
                pltpu.make_async_copy(
                    og_v, out_hbm.at[pl.ds(u * (_UV // 8), _UV // 8)],
                    osem).wait()

            _shuffle(tab_v, og_v, _UV)
            pltpu.async_copy(
                og_v, out_hbm.at[pl.ds(u * (_UV // 8), _UV // 8)], osem)

        return 0

    lax.fori_loop(0, (_NUNIT + _NW - 1) // _NW, unit, 0)
    pltpu.make_async_copy(og_v, out_hbm.at[pl.ds(0, _UV // 8)], osem).wait()

    @pl.when(wid == 1)
    def _():
        pltpu.sync_copy(tab_hbm.at[:, pl.ds(_TAIL0, _MID)],
                        tab_v.at[:, pl.ds(0, _MID)])
        _shuffle(tab_v, og_v, _MID)
        pltpu.sync_copy(og_v.at[pl.ds(0, _MID // 8)],
                        out_hbm.at[pl.ds(_TAIL0 // 8, _MID // 8)])

    @pl.when(wid == 0)
    def _():
        pltpu.sync_copy(tail_hbm, tab_v.at[:, pl.ds(0, 128)])
        _shuffle(tab_v, og_v, 128)
        pltpu.sync_copy(og_v.at[pl.ds(0, 8)],
                        out_hbm.at[pl.ds(_TAIL1 // 8, 8)])


@functools.partial(
    pl.kernel,
    out_type=jax.ShapeDtypeStruct((_B, _EMBED), jnp.float32),
    mesh=_mesh,
    scratch_types=[
        pltpu.VMEM((_CH,), jnp.int32),
        pltpu.VMEM((_CH, _EMBED), jnp.float32),
        pltpu.SemaphoreType.DMA,
    ],
    compiler_params=pltpu.CompilerParams(use_tc_tiling_on_sc=False),
)
def _gather_kernel(idx_hbm, table_hbm, out_hbm, idx_v, rows_v, sem):
    wid = lax.axis_index("s") * 2 + lax.axis_index("c")
    base = wid * _B_PER_W
    for g in range(_NCHUNK):
        off = base + g * _CH
        pltpu.sync_copy(idx_hbm.at[pl.ds(off, _CH)], idx_v)
        pltpu.async_copy(table_hbm.at[idx_v], rows_v, sem).wait()
        pltpu.sync_copy(rows_v, out_hbm.at[pl.ds(off, _CH)])


@functools.partial(
    pl.kernel,
    out_type=jax.ShapeDtypeStruct((_N_FIELDS, _EMBED, _BATCH), jnp.float32),
    mesh=_mesh,
    scratch_types=[
        pltpu.VMEM((128, 128), jnp.float32),
        pltpu.VMEM((_EMBED, 1024), jnp.float32),
    ],
    compiler_params=pltpu.CompilerParams(needs_layout_passes=False),
)
def _retile_kernel(rows_hbm, out_hbm, tin_v, och_v):
    i16 = lax.iota(jnp.int32, 16)
    hi = i16 >> 3
    si = (i16 & 7) * _EMBED
    wid = lax.axis_index("s") * 2 + lax.axis_index("c")

    def unit(c, _):
        f = c // 16
        q = c % 16
        pltpu.sync_copy(rows_hbm.at[f, pl.ds(q * 128, 128), :], tin_v)
        for j in range(_EMBED):
            colj = si + j

            @plsc.parallel_loop(0, 64, unroll=4)
            def _(k):
                rowk = hi + 2 * k
                och_v[j, pl.ds(k * 16, 16)] = plsc.load_gather(
                    tin_v, [rowk, colj])
        pltpu.sync_copy(och_v, out_hbm.at[f, :, pl.ds(q * 1024, 1024)])
        return 0

    lax.fori_loop(wid * 13, (wid + 1) * 13, unit, 0)


def kernel(index, cluster_index):
    table_t = jnp.swapaxes(cluster_index, 0, 1)
    tail = jnp.zeros((16, 128), jnp.float32)
    tail = lax.dynamic_update_slice(
        tail, lax.slice(table_t, (0, _TAIL1), (16, _VOCAB)), (0, 0))
    table_g = _repack_kernel(table_t, tail)
    table_v = jnp.reshape(table_g, (_VOCAB, _EMBED))
    flat_idx = jnp.swapaxes(index, 0, 1).reshape(-1)
    rows = _gather_kernel(flat_idx, table_v)
    rows3 = jnp.reshape(rows, (_N_FIELDS, _BATCH // 8, 128))
    out3 = _retile_kernel(rows3)
    return jnp.transpose(out3, (2, 0, 1))
